# baseline (speedup 1.0000x reference)
import functools
import math

import jax
import jax.numpy as jnp
from jax import lax
from jax.experimental import pallas as pl
from jax.experimental.pallas import tpu as pltpu
from jax.experimental.pallas import tpu_sc as plsc

D_MODEL = 64
SCALE = math.sqrt(D_MODEL)

_info = plsc.get_sparse_core_info()
_NC, _NS = _info.num_cores, _info.num_subcores
_NW = _NC * _NS
_BW = 128
_NBUF = 4


@jax.jit
def _embed_lookup(x4, table):
    st_n, bt_n, _, _ = x4.shape
    n_s = st_n * 8
    assert bt_n == _NW

    mesh = plsc.VectorSubcoreMesh(core_axis_name="c", subcore_axis_name="s")

    @functools.partial(
        pl.kernel,
        mesh=mesh,
        out_type=jax.ShapeDtypeStruct((n_s, 8, _NW, 8, _BW), jnp.float32),
        scratch_types=[
            pltpu.VMEM((st_n, 8, _BW), jnp.int32),
            [pltpu.VMEM((_BW, D_MODEL), jnp.float32)] * _NBUF,
            [pltpu.VMEM((8, 8, _BW), jnp.float32)] * _NBUF,
            [pltpu.SemaphoreType.DMA] * _NBUF,
            [pltpu.SemaphoreType.DMA] * _NBUF,
        ],
        compiler_params=pltpu.CompilerParams(use_tc_tiling_on_sc=False,
                                               needs_layout_passes=False),
    )
    def k(x4_hbm, table_hbm, out_hbm, idx_v, rows, tblk, gsems, osems):
        wid = lax.axis_index("s") * _NC + lax.axis_index("c")
        pltpu.sync_copy(x4_hbm.at[:, wid], idx_v)

        viotas = [lax.iota(jnp.int32, 16) + 16 * j for j in range(8)]

        def issue_gather(i, b):
            pltpu.async_copy(
                table_hbm.at[idx_v.at[lax.div(i, 8), lax.rem(i, 8)]],
                rows[b], gsems[b])

        def wait_gather(b):
            pltpu.make_async_copy(table_hbm.at[idx_v.at[0, 0]],
                                  rows[b], gsems[b]).wait()

        def issue_out(i, b):
            pltpu.async_copy(tblk[b], out_hbm.at[i, :, wid], osems[b])

        def wait_out(b):
            pltpu.make_async_copy(tblk[b], out_hbm.at[0, :, 0],
                                  osems[b]).wait()

        def transpose_scale(b):
            rb, tb = rows[b], tblk[b]
            def d_body(d, carry):
                dvec = jnp.full((16,), 0, jnp.int32) + d
                dt, dsub = lax.div(d, 8), lax.rem(d, 8)
                for j in range(8):
                    v = plsc.load_gather(rb, [viotas[j], dvec])
                    tb[dt, dsub, pl.ds(16 * j, 16)] = v * SCALE
                return carry
            lax.fori_loop(0, D_MODEL, d_body, 0)

        def step_b(i, b, drain):
            wait_gather(b)
            if drain:
                wait_out(b)
            transpose_scale(b)
            issue_out(i, b)

        issue_gather(0, 0)
        issue_gather(1, 1)
        issue_gather(2, 2)
        step_b(0, 0, False)
        issue_gather(3, 3)
        step_b(1, 1, False)
        issue_gather(4, 0)
        step_b(2, 2, False)
        issue_gather(5, 1)
        step_b(3, 3, False)

        def loop_body(kk, carry):
            i0 = 4 * kk
            for m in range(4):
                i = i0 + m
                issue_gather(i + 2, (m + 2) % 4)
                step_b(i, m, True)
            return carry

        lax.fori_loop(1, n_s // 4 - 1, loop_body, 0)

        nl = n_s - 4
        issue_gather(nl + 2, 2)
        step_b(nl + 0, 0, True)
        issue_gather(nl + 3, 3)
        step_b(nl + 1, 1, True)
        step_b(nl + 2, 2, True)
        step_b(nl + 3, 3, True)
        for b in range(_NBUF):
            wait_out(b)

    return k(x4, table)


def kernel(x, table):
    b, s = x.shape
    xt = jnp.transpose(x, (1, 0)).astype(jnp.int32)
    x4 = jnp.transpose(xt.reshape(s // 8, 8, b // _BW, _BW), (0, 2, 1, 3))
    out5 = _embed_lookup(x4, table)
    out = jnp.transpose(out5, (2, 4, 0, 1, 3)).reshape(b, s, D_MODEL)
    return out


# speedup vs baseline: 1.5449x; 1.5449x over previous
import functools
import math

import jax
import jax.numpy as jnp
from jax import lax
from jax.experimental import pallas as pl
from jax.experimental.pallas import tpu as pltpu
from jax.experimental.pallas import tpu_sc as plsc

D_MODEL = 64
SCALE = math.sqrt(D_MODEL)

_info = plsc.get_sparse_core_info()
_NC, _NS = _info.num_cores, _info.num_subcores
_NW = _NC * _NS
_BW = 128
_NBUF = 4


@jax.jit
def _embed_lookup(x4, table):
    st_n, bt_n, _, _ = x4.shape
    n_s = st_n * 8
    assert bt_n == _NW

    mesh = plsc.VectorSubcoreMesh(core_axis_name="c", subcore_axis_name="s")

    @functools.partial(
        pl.kernel,
        mesh=mesh,
        out_type=jax.ShapeDtypeStruct((n_s, 8, _NW, 8, _BW), jnp.float32),
        scratch_types=[
            pltpu.VMEM((st_n, 8, _BW), jnp.int32),
            [pltpu.VMEM((_BW, D_MODEL), jnp.float32)] * _NBUF,
            [pltpu.VMEM((8, 8, _BW), jnp.float32)] * _NBUF,
            [pltpu.SemaphoreType.DMA] * _NBUF,
            [pltpu.SemaphoreType.DMA] * _NBUF,
        ],
        compiler_params=pltpu.CompilerParams(use_tc_tiling_on_sc=False,
                                               needs_layout_passes=False),
    )
    def k(x4_hbm, table_hbm, out_hbm, idx_v, rows, tblk, gsems, osems):
        wid = lax.axis_index("s") * _NC + lax.axis_index("c")
        pltpu.sync_copy(x4_hbm.at[:, wid], idx_v)

        viotas = [lax.iota(jnp.int32, 16) + 16 * j for j in range(8)]

        def issue_gather(i, b):
            pltpu.async_copy(
                table_hbm.at[idx_v.at[lax.div(i, 8), lax.rem(i, 8)]],
                rows[b], gsems[b])

        def wait_gather(b):
            pltpu.make_async_copy(table_hbm.at[idx_v.at[0, 0]],
                                  rows[b], gsems[b]).wait()

        def issue_out(i, b):
            pltpu.async_copy(tblk[b], out_hbm.at[i, :, wid], osems[b])

        def wait_out(b):
            pltpu.make_async_copy(tblk[b], out_hbm.at[0, :, 0],
                                  osems[b]).wait()

        def transpose_scale(b):
            rb, tb = rows[b], tblk[b]
            @plsc.parallel_loop(0, D_MODEL, unroll=4)
            def d_body(d):
                dvec = jnp.full((16,), 0, jnp.int32) + d
                dt, dsub = lax.div(d, 8), lax.rem(d, 8)
                for j in range(8):
                    v = plsc.load_gather(rb, [viotas[j], dvec])
                    tb[dt, dsub, pl.ds(16 * j, 16)] = v * SCALE

        def step_b(i, b, drain):
            wait_gather(b)
            if drain:
                wait_out(b)
            transpose_scale(b)
            issue_out(i, b)

        issue_gather(0, 0)
        issue_gather(1, 1)
        issue_gather(2, 2)
        step_b(0, 0, False)
        issue_gather(3, 3)
        step_b(1, 1, False)
        issue_gather(4, 0)
        step_b(2, 2, False)
        issue_gather(5, 1)
        step_b(3, 3, False)

        def loop_body(kk, carry):
            i0 = 4 * kk
            for m in range(4):
                i = i0 + m
                issue_gather(i + 2, (m + 2) % 4)
                step_b(i, m, True)
            return carry

        lax.fori_loop(1, n_s // 4 - 1, loop_body, 0)

        nl = n_s - 4
        issue_gather(nl + 2, 2)
        step_b(nl + 0, 0, True)
        issue_gather(nl + 3, 3)
        step_b(nl + 1, 1, True)
        step_b(nl + 2, 2, True)
        step_b(nl + 3, 3, True)
        for b in range(_NBUF):
            wait_out(b)

    return k(x4, table)


def kernel(x, table):
    b, s = x.shape
    xt = jnp.transpose(x, (1, 0)).astype(jnp.int32)
    x4 = jnp.transpose(xt.reshape(s // 8, 8, b // _BW, _BW), (0, 2, 1, 3))
    out5 = _embed_lookup(x4, table)
    out = jnp.transpose(out5, (2, 4, 0, 1, 3)).reshape(b, s, D_MODEL)
    return out


# speedup vs baseline: 2.3403x; 1.5149x over previous
import functools
import math

import jax
import jax.numpy as jnp
from jax import lax
from jax.experimental import pallas as pl
from jax.experimental.pallas import tpu as pltpu
from jax.experimental.pallas import tpu_sc as plsc

D_MODEL = 64
SCALE = math.sqrt(D_MODEL)

_info = plsc.get_sparse_core_info()
_NC, _NS = _info.num_cores, _info.num_subcores
_NW = _NC * _NS
_BW = 128
_NBUF = 4


@jax.jit
def _embed_lookup(x4, table):
    st_n, bt_n, _, _ = x4.shape
    n_s = st_n * 8
    assert bt_n == _NW

    mesh = plsc.VectorSubcoreMesh(core_axis_name="c", subcore_axis_name="s")

    @functools.partial(
        pl.kernel,
        mesh=mesh,
        out_type=jax.ShapeDtypeStruct((n_s, 8, _NW, 8, _BW), jnp.float32),
        scratch_types=[
            pltpu.VMEM((st_n, 8, _BW), jnp.int32),
            [pltpu.VMEM((_BW, D_MODEL), jnp.float32)] * _NBUF,
            [pltpu.VMEM((D_MODEL, _BW), jnp.float32)] * _NBUF,
            [pltpu.SemaphoreType.DMA] * _NBUF,
            [pltpu.SemaphoreType.DMA] * _NBUF,
        ],
        compiler_params=pltpu.CompilerParams(use_tc_tiling_on_sc=False,
                                               needs_layout_passes=False),
    )
    def k(x4_hbm, table_hbm, out_hbm, idx_v, rows, tblk, gsems, osems):
        wid = lax.axis_index("s") * _NC + lax.axis_index("c")
        pltpu.sync_copy(x4_hbm.at[:, wid], idx_v)

        viota = lax.iota(jnp.int32, 16)
        vrots = [lax.rem(viota + t, 16) for t in range(16)]

        def issue_gather(i, b):
            pltpu.async_copy(
                table_hbm.at[idx_v.at[lax.div(i, 8), lax.rem(i, 8)]],
                rows[b], gsems[b])

        def wait_gather(b):
            pltpu.make_async_copy(table_hbm.at[idx_v.at[0, 0]],
                                  rows[b], gsems[b]).wait()

        def issue_out(i, b):
            for dt in range(8):
                pltpu.async_copy(tblk[b].at[pl.ds(8 * dt, 8)],
                                 out_hbm.at[i, dt, wid], osems[b])

        def wait_out(b):
            for dt in range(8):
                pltpu.make_async_copy(tblk[b].at[pl.ds(8 * dt, 8)],
                                      out_hbm.at[0, 0, 0], osems[b]).wait()

        def transpose_scale(b):
            rb, tb = rows[b], tblk[b]
            @plsc.parallel_loop(0, 32, unroll=2)
            def blk_body(i):
                j, d0 = lax.rem(i, 8), lax.div(i, 8) * 16
                ib = viota + 16 * j
                for t in range(16):
                    idv = vrots[t] + d0
                    v = plsc.load_gather(rb, [ib, idv])
                    plsc.store_scatter(tb, [idv, ib], v * SCALE)

        def step_b(i, b, drain):
            wait_gather(b)
            if drain:
                wait_out(b)
            transpose_scale(b)
            issue_out(i, b)

        issue_gather(0, 0)
        issue_gather(1, 1)
        issue_gather(2, 2)
        step_b(0, 0, False)
        issue_gather(3, 3)
        step_b(1, 1, False)
        issue_gather(4, 0)
        step_b(2, 2, False)
        issue_gather(5, 1)
        step_b(3, 3, False)

        def loop_body(kk, carry):
            i0 = 4 * kk
            for m in range(4):
                i = i0 + m
                issue_gather(i + 2, (m + 2) % 4)
                step_b(i, m, True)
            return carry

        lax.fori_loop(1, n_s // 4 - 1, loop_body, 0)

        nl = n_s - 4
        issue_gather(nl + 2, 2)
        step_b(nl + 0, 0, True)
        issue_gather(nl + 3, 3)
        step_b(nl + 1, 1, True)
        step_b(nl + 2, 2, True)
        step_b(nl + 3, 3, True)
        for b in range(_NBUF):
            wait_out(b)

    return k(x4, table)


def kernel(x, table):
    b, s = x.shape
    xt = jnp.transpose(x, (1, 0)).astype(jnp.int32)
    x4 = jnp.transpose(xt.reshape(s // 8, 8, b // _BW, _BW), (0, 2, 1, 3))
    out5 = _embed_lookup(x4, table)
    out = jnp.transpose(out5, (2, 4, 0, 1, 3)).reshape(b, s, D_MODEL)
    return out
